# baseline (speedup 1.0000x reference)
import jax
import jax.numpy as jnp
from jax.experimental import pallas as pl


def _gcn_conv(x, edge_index, edge_weight, W, b, n_nodes):
    src = edge_index[0]
    dst = edge_index[1]
    loop = jnp.arange(n_nodes, dtype=edge_index.dtype)
    src2 = jnp.concatenate([src, loop])
    dst2 = jnp.concatenate([dst, loop])
    ew2 = jnp.concatenate([edge_weight, jnp.ones((n_nodes,), dtype=edge_weight.dtype)])
    deg = jax.ops.segment_sum(ew2, dst2, num_segments=n_nodes)
    deg_inv_sqrt = jnp.where(deg > 0, jax.lax.rsqrt(jnp.where(deg > 0, deg, 1.0)), 0.0)
    norm = deg_inv_sqrt[src2] * ew2 * deg_inv_sqrt[dst2]
    h = x @ W
    msg = h[src2] * norm[:, None]
    out = jax.ops.segment_sum(msg, dst2, num_segments=n_nodes)
    return out + b


def kernel(x, edge_index, edge_weight, W1, b1, W2, b2, W_mu, b_mu, W_ls, b_ls):
    n = x.shape[0]
    ones = jnp.ones((edge_index.shape[1],), dtype=x.dtype)
    h = _gcn_conv(x, edge_index, edge_weight, W1, b1, n)
    h = jax.nn.relu(h)
    h = _gcn_conv(h, edge_index, edge_weight, W2, b2, n)
    mu = _gcn_conv(h, edge_index, ones, W_mu, b_mu, n)
    logstd = _gcn_conv(h, edge_index, ones, W_ls, b_ls, n)
    return (mu, logstd)


# speedup vs baseline: 20.1070x; 20.1070x over previous
import functools

import jax
import jax.numpy as jnp
from jax import lax
from jax.experimental import pallas as pl
from jax.experimental.pallas import tpu as pltpu, tpu_sc as plsc

N = 10000
NPAD = 10240
E = 320000
NC = 2
NS = 16
NW = NC * NS
EPW = E // NW
K = 80
NCHUNK = EPW // K
RPW = NPAD // NS
RBLK = K
NRB = RPW // RBLK

_sc_params = pltpu.CompilerParams(use_tc_tiling_on_sc=False)

_mesh = plsc.VectorSubcoreMesh(core_axis_name="c", subcore_axis_name="s",
                               num_cores=NC, num_subcores=NS)


def _zero_block(ref, nrows, ncol16):
    zero16 = jnp.zeros((16,), jnp.float32)

    def body(i, _):
        for q in range(ncol16):
            ref[i, pl.ds(q * 16, 16)] = zero16
        return 0

    lax.fori_loop(0, nrows, body, 0)


@functools.partial(
    pl.kernel,
    out_type=[jax.ShapeDtypeStruct((NC, NPAD, 16), jnp.float32),
              jax.ShapeDtypeStruct((NC, NPAD, 16), jnp.float32)],
    mesh=_mesh,
    compiler_params=_sc_params,
    scratch_types=[
        pltpu.VMEM((NCHUNK, K), jnp.int32),
        pltpu.VMEM((NCHUNK, K), jnp.float32),
        pltpu.VMEM((K, 16), jnp.float32),
        pltpu.VMEM((K, 16), jnp.float32),
        pltpu.VMEM((RPW, 16), jnp.float32),
        pltpu.VMEM_SHARED((NPAD, 16), jnp.float32),
        pltpu.VMEM_SHARED((NPAD, 16), jnp.float32),
    ],
)
def _deg_kernel(dst_hbm, ew_hbm, outw_hbm, out1_hbm,
                dst_v, ew_v, buf_w, buf_1, zblk, hist_w, hist_1):
    c = lax.axis_index("c")
    s = lax.axis_index("s")
    chunk = c * NS + s
    pltpu.sync_copy(dst_hbm.at[chunk], dst_v)
    pltpu.sync_copy(ew_hbm.at[chunk], ew_v)

    _zero_block(zblk, RPW, 1)
    pltpu.sync_copy(zblk, hist_w.at[pl.ds(s * RPW, RPW)])
    pltpu.sync_copy(zblk, hist_1.at[pl.ds(s * RPW, RPW)])

    ones_f = jnp.ones((16,), jnp.float32)

    def ones_body(i, _):
        buf_1[i, :] = ones_f
        return 0

    lax.fori_loop(0, K, ones_body, 0)
    plsc.subcore_barrier()

    def chunk_body(j, _):
        def grp_body(g, _):
            wv = ew_v[j, pl.ds(g * 16, 16)]
            for l in range(16):
                buf_w[g * 16 + l, :] = jnp.broadcast_to(wv[l], (16,))
            return 0
        lax.fori_loop(0, K // 16, grp_body, 0)
        pltpu.sync_copy(buf_w, hist_w.at[dst_v.at[j]], add=True)
        pltpu.sync_copy(buf_1, hist_1.at[dst_v.at[j]], add=True)
        return 0

    lax.fori_loop(0, NCHUNK, chunk_body, 0)
    plsc.subcore_barrier()

    pltpu.sync_copy(hist_w.at[pl.ds(s * RPW, RPW)], zblk)
    pltpu.sync_copy(zblk, outw_hbm.at[c, pl.ds(s * RPW, RPW)])
    pltpu.sync_copy(hist_1.at[pl.ds(s * RPW, RPW)], zblk)
    pltpu.sync_copy(zblk, out1_hbm.at[c, pl.ds(s * RPW, RPW)])


def _make_agg_kernel(weighted):
    scratch = [
        pltpu.VMEM((NCHUNK, K), jnp.int32),
        pltpu.VMEM((NCHUNK, K), jnp.int32),
        pltpu.VMEM((NCHUNK, K), jnp.float32),
        pltpu.VMEM((K, 128), jnp.float32),
        pltpu.VMEM_SHARED((NPAD, 128), jnp.float32),
        pltpu.SemaphoreType.DMA,
    ]

    def body(table_hbm, src_hbm, dst_hbm, ew_hbm, out_hbm,
             src_v, dst_v, ew_v, rows_v, acc, sem):
        c = lax.axis_index("c")
        s = lax.axis_index("s")
        chunk = c * NS + s
        pltpu.sync_copy(src_hbm.at[chunk], src_v)
        pltpu.sync_copy(dst_hbm.at[chunk], dst_v)
        if weighted:
            pltpu.sync_copy(ew_hbm.at[chunk], ew_v)

        _zero_block(rows_v, RBLK, 8)
        for t in range(NRB):
            pltpu.sync_copy(rows_v, acc.at[pl.ds(s * RPW + t * RBLK, RBLK)])
        plsc.subcore_barrier()

        def chunk_body(j, _):
            pltpu.async_copy(table_hbm.at[src_v.at[j]], rows_v, sem).wait()
            if weighted:
                def grp_body(g, _):
                    wv = ew_v[j, pl.ds(g * 16, 16)]
                    for l in range(16):
                        w = wv[l]
                        r = g * 16 + l
                        for q in range(8):
                            sl = pl.ds(q * 16, 16)
                            rows_v[r, sl] = rows_v[r, sl] * w
                    return 0
                lax.fori_loop(0, K // 16, grp_body, 0)
            pltpu.sync_copy(rows_v, acc.at[dst_v.at[j]], add=True)
            return 0

        lax.fori_loop(0, NCHUNK, chunk_body, 0)
        plsc.subcore_barrier()

        for t in range(NRB):
            pltpu.sync_copy(acc.at[pl.ds(s * RPW + t * RBLK, RBLK)], rows_v)
            pltpu.sync_copy(rows_v, out_hbm.at[c, pl.ds(s * RPW + t * RBLK, RBLK)])

    return pl.kernel(
        body,
        out_type=jax.ShapeDtypeStruct((NC, NPAD, 128), jnp.float32),
        mesh=_mesh,
        compiler_params=_sc_params,
        scratch_types=scratch,
    )


_agg_w = _make_agg_kernel(True)
_agg_1 = _make_agg_kernel(False)


_BR = 1000
_NB = N // _BR


def _scales_body(histw_ref, hist1_ref, x_ref, xs_ref, aux_ref):
    hw = histw_ref[...]
    h1 = hist1_ref[...]
    degw = hw[0, :, 0] + hw[1, :, 0] + 1.0
    deg1 = h1[0, :, 0] + h1[1, :, 0] + 1.0
    disw = jnp.where(degw > 0, lax.rsqrt(degw), 0.0)
    dis1 = jnp.where(deg1 > 0, lax.rsqrt(deg1), 0.0)
    aux_ref[...] = jnp.stack([disw, disw * disw, dis1, dis1 * dis1], axis=1)
    xs_ref[...] = x_ref[...] * disw[:, None]


def _scales_call(histw, hist1, x):
    return pl.pallas_call(
        _scales_body,
        grid=(_NB,),
        in_specs=[
            pl.BlockSpec((NC, _BR, 16), lambda i: (0, i, 0)),
            pl.BlockSpec((NC, _BR, 16), lambda i: (0, i, 0)),
            pl.BlockSpec((_BR, 128), lambda i: (i, 0)),
        ],
        out_specs=[
            pl.BlockSpec((_BR, 128), lambda i: (i, 0)),
            pl.BlockSpec((_BR, 4), lambda i: (i, 0)),
        ],
        out_shape=[
            jax.ShapeDtypeStruct((N, 128), jnp.float32),
            jax.ShapeDtypeStruct((N, 4), jnp.float32),
        ],
    )(histw, hist1, x)


def _make_conv_body(col, relu, scale_col):
    def body(p_ref, xin_ref, aux_ref, w_ref, b_ref, out_ref, *scaled_ref):
        aux = aux_ref[...]
        t = ((p_ref[0] + p_ref[1]) * aux[:, col][:, None]
             + xin_ref[...] * aux[:, col + 1][:, None])
        o = jnp.dot(t, w_ref[...], preferred_element_type=jnp.float32) + b_ref[0, :]
        if relu:
            o = jnp.maximum(o, 0.0)
        out_ref[...] = o
        if scale_col is not None:
            scaled_ref[0][...] = o * aux[:, scale_col][:, None]

    return body


def _conv_call(p, xin, aux, w, b, col, relu, scale_col):
    kout = w.shape[1]
    out_shape = [jax.ShapeDtypeStruct((N, kout), jnp.float32)]
    out_specs = [pl.BlockSpec((_BR, kout), lambda i: (i, 0))]
    if scale_col is not None:
        out_shape.append(jax.ShapeDtypeStruct((N, kout), jnp.float32))
        out_specs.append(pl.BlockSpec((_BR, kout), lambda i: (i, 0)))
    res = pl.pallas_call(
        _make_conv_body(col, relu, scale_col),
        grid=(_NB,),
        in_specs=[
            pl.BlockSpec((NC, _BR, 128), lambda i: (0, i, 0)),
            pl.BlockSpec((_BR, 128), lambda i: (i, 0)),
            pl.BlockSpec((_BR, 4), lambda i: (i, 0)),
            pl.BlockSpec((128, kout), lambda i: (0, 0)),
            pl.BlockSpec((1, kout), lambda i: (0, 0)),
        ],
        out_specs=out_specs,
        out_shape=out_shape,
    )(p, xin, aux, w, b.reshape(1, kout))
    return res if scale_col is not None else (res[0], None)


def _heads_body(p_ref, xin_ref, aux_ref, wmu_ref, bmu_ref, wls_ref, bls_ref,
                mu_ref, ls_ref):
    aux = aux_ref[...]
    agg = ((p_ref[0] + p_ref[1]) * aux[:, 2][:, None]
           + xin_ref[...] * aux[:, 3][:, None])
    mu_ref[...] = jnp.dot(agg, wmu_ref[...],
                          preferred_element_type=jnp.float32) + bmu_ref[0, :]
    ls_ref[...] = jnp.dot(agg, wls_ref[...],
                          preferred_element_type=jnp.float32) + bls_ref[0, :]


def _heads_call(p, xin, aux, wmu, bmu, wls, bls):
    kout = wmu.shape[1]
    return pl.pallas_call(
        _heads_body,
        grid=(_NB,),
        in_specs=[
            pl.BlockSpec((NC, _BR, 128), lambda i: (0, i, 0)),
            pl.BlockSpec((_BR, 128), lambda i: (i, 0)),
            pl.BlockSpec((_BR, 4), lambda i: (i, 0)),
            pl.BlockSpec((128, kout), lambda i: (0, 0)),
            pl.BlockSpec((1, kout), lambda i: (0, 0)),
            pl.BlockSpec((128, kout), lambda i: (0, 0)),
            pl.BlockSpec((1, kout), lambda i: (0, 0)),
        ],
        out_specs=[
            pl.BlockSpec((_BR, kout), lambda i: (i, 0)),
            pl.BlockSpec((_BR, kout), lambda i: (i, 0)),
        ],
        out_shape=[
            jax.ShapeDtypeStruct((N, kout), jnp.float32),
            jax.ShapeDtypeStruct((N, kout), jnp.float32),
        ],
    )(p, xin, aux, wmu, bmu.reshape(1, kout), wls, bls.reshape(1, kout))


def kernel(x, edge_index, edge_weight, W1, b1, W2, b2, W_mu, b_mu, W_ls, b_ls):
    ei = edge_index.astype(jnp.int32)
    src3 = ei[0].reshape(NW, NCHUNK, K)
    dst3 = ei[1].reshape(NW, NCHUNK, K)
    ew3 = edge_weight.reshape(NW, NCHUNK, K)

    histw, hist1 = _deg_kernel(dst3, ew3)
    xs0, aux = _scales_call(histw, hist1, x)

    p1 = _agg_w(xs0, src3, dst3, ew3)
    h1, h1s = _conv_call(p1, x, aux, W1, b1, col=0, relu=True, scale_col=0)

    p2 = _agg_w(h1s, src3, dst3, ew3)
    h2, h2s = _conv_call(p2, h1, aux, W2, b2, col=0, relu=False, scale_col=2)

    p3 = _agg_1(h2s, src3, dst3, ew3)
    mu, ls = _heads_call(p3, h2, aux, W_mu, b_mu, W_ls, b_ls)
    return (mu, ls)


# speedup vs baseline: 31.5015x; 1.5667x over previous
import functools

import jax
import jax.numpy as jnp
from jax import lax
from jax.experimental import pallas as pl
from jax.experimental.pallas import tpu as pltpu, tpu_sc as plsc

N = 10000
NPAD = 10240
E = 320000
NC = 2
NS = 16
NW = NC * NS
EPW = E // NW
K = 80
NCHUNK = EPW // K
RPW = NPAD // NS
RBLK = K
NRB = RPW // RBLK

_sc_params = pltpu.CompilerParams(use_tc_tiling_on_sc=False)

_mesh = plsc.VectorSubcoreMesh(core_axis_name="c", subcore_axis_name="s",
                               num_cores=NC, num_subcores=NS)


def _zero_block(ref, nrows, ncol16):
    zero16 = jnp.zeros((16,), jnp.float32)

    def body(i, _):
        for q in range(ncol16):
            ref[i, pl.ds(q * 16, 16)] = zero16
        return 0

    lax.fori_loop(0, nrows, body, 0)


@functools.partial(
    pl.kernel,
    out_type=[jax.ShapeDtypeStruct((NC, NPAD, 16), jnp.float32),
              jax.ShapeDtypeStruct((NC, NPAD, 16), jnp.float32)],
    mesh=_mesh,
    compiler_params=_sc_params,
    scratch_types=[
        pltpu.VMEM((NCHUNK, K), jnp.int32),
        pltpu.VMEM((NCHUNK, K), jnp.float32),
        pltpu.VMEM((K, 16), jnp.float32),
        pltpu.VMEM((K, 16), jnp.float32),
        pltpu.VMEM((RPW, 16), jnp.float32),
        pltpu.VMEM_SHARED((NPAD, 16), jnp.float32),
        pltpu.VMEM_SHARED((NPAD, 16), jnp.float32),
    ],
)
def _deg_kernel(dst_hbm, ew_hbm, outw_hbm, out1_hbm,
                dst_v, ew_v, buf_w, buf_1, zblk, hist_w, hist_1):
    c = lax.axis_index("c")
    s = lax.axis_index("s")
    chunk = c * NS + s
    pltpu.sync_copy(dst_hbm.at[chunk], dst_v)
    pltpu.sync_copy(ew_hbm.at[chunk], ew_v)

    _zero_block(zblk, RPW, 1)
    pltpu.sync_copy(zblk, hist_w.at[pl.ds(s * RPW, RPW)])
    pltpu.sync_copy(zblk, hist_1.at[pl.ds(s * RPW, RPW)])

    ones_f = jnp.ones((16,), jnp.float32)

    def ones_body(i, _):
        buf_1[i, :] = ones_f
        return 0

    lax.fori_loop(0, K, ones_body, 0)
    plsc.subcore_barrier()

    def chunk_body(j, _):
        def grp_body(g, _):
            wv = ew_v[j, pl.ds(g * 16, 16)]
            for l in range(16):
                buf_w[g * 16 + l, :] = jnp.broadcast_to(wv[l], (16,))
            return 0
        lax.fori_loop(0, K // 16, grp_body, 0)
        pltpu.sync_copy(buf_w, hist_w.at[dst_v.at[j]], add=True)
        pltpu.sync_copy(buf_1, hist_1.at[dst_v.at[j]], add=True)
        return 0

    lax.fori_loop(0, NCHUNK, chunk_body, 0)
    plsc.subcore_barrier()

    pltpu.sync_copy(hist_w.at[pl.ds(s * RPW, RPW)], zblk)
    pltpu.sync_copy(zblk, outw_hbm.at[c, pl.ds(s * RPW, RPW)])
    pltpu.sync_copy(hist_1.at[pl.ds(s * RPW, RPW)], zblk)
    pltpu.sync_copy(zblk, out1_hbm.at[c, pl.ds(s * RPW, RPW)])


def _make_agg_kernel(weighted):
    scratch = [
        pltpu.VMEM((NCHUNK, K), jnp.int32),
        pltpu.VMEM((NCHUNK, K), jnp.int32),
        pltpu.VMEM((K, 128), jnp.float32),
        pltpu.VMEM((K, 128), jnp.float32),
        pltpu.VMEM((1, K), jnp.float32),
        pltpu.VMEM((1, K), jnp.float32),
        pltpu.VMEM_SHARED((NPAD, 128), jnp.float32),
        pltpu.SemaphoreType.DMA,
        pltpu.SemaphoreType.DMA,
        pltpu.SemaphoreType.DMA,
        pltpu.SemaphoreType.DMA,
    ]

    def body(table_hbm, src_hbm, dst_hbm, ew_hbm, out_hbm,
             src_v, dst_v, rows_a, rows_b, ew_a, ew_b, acc,
             sem_ra, sem_rb, sem_wa, sem_wb):
        c = lax.axis_index("c")
        s = lax.axis_index("s")
        chunk = c * NS + s
        pltpu.sync_copy(src_hbm.at[chunk], src_v)
        pltpu.sync_copy(dst_hbm.at[chunk], dst_v)

        _zero_block(rows_a, RBLK, 8)
        for t in range(NRB):
            pltpu.sync_copy(rows_a, acc.at[pl.ds(s * RPW + t * RBLK, RBLK)])
        plsc.subcore_barrier()

        def fetch(j, rows, ew, sem_r, sem_w):
            pltpu.async_copy(table_hbm.at[src_v.at[j]], rows, sem_r)
            if weighted:
                pltpu.async_copy(ew_hbm.at[chunk].at[pl.ds(j, 1)], ew, sem_w)

        def wait(rows, ew, sem_r, sem_w):
            pltpu.make_async_copy(table_hbm.at[src_v.at[0]], rows, sem_r).wait()
            if weighted:
                pltpu.make_async_copy(ew_hbm.at[0].at[pl.ds(0, 1)], ew,
                                      sem_w).wait()

        def process(j, rows, ew):
            if weighted:
                def grp_body(g, _):
                    wv = ew[0, pl.ds(g * 16, 16)]
                    for l in range(16):
                        w = wv[l]
                        r = g * 16 + l
                        for q in range(8):
                            sl = pl.ds(q * 16, 16)
                            rows[r, sl] = rows[r, sl] * w
                    return 0
                lax.fori_loop(0, K // 16, grp_body, 0)
            pltpu.sync_copy(rows, acc.at[dst_v.at[j]], add=True)

        fetch(0, rows_a, ew_a, sem_ra, sem_wa)

        def pair_body(i, _):
            c0 = 2 * i
            fetch(c0 + 1, rows_b, ew_b, sem_rb, sem_wb)
            wait(rows_a, ew_a, sem_ra, sem_wa)
            process(c0, rows_a, ew_a)
            fetch(c0 + 2, rows_a, ew_a, sem_ra, sem_wa)
            wait(rows_b, ew_b, sem_rb, sem_wb)
            process(c0 + 1, rows_b, ew_b)
            return 0

        lax.fori_loop(0, (NCHUNK - 1) // 2, pair_body, 0)
        wait(rows_a, ew_a, sem_ra, sem_wa)
        process(NCHUNK - 1, rows_a, ew_a)
        plsc.subcore_barrier()

        for t in range(NRB):
            pltpu.sync_copy(acc.at[pl.ds(s * RPW + t * RBLK, RBLK)], rows_a)
            pltpu.sync_copy(rows_a, out_hbm.at[c, pl.ds(s * RPW + t * RBLK, RBLK)])

    return pl.kernel(
        body,
        out_type=jax.ShapeDtypeStruct((NC, NPAD, 128), jnp.float32),
        mesh=_mesh,
        compiler_params=_sc_params,
        scratch_types=scratch,
    )


_agg_w = _make_agg_kernel(True)
_agg_1 = _make_agg_kernel(False)


_BR = 1000
_NB = N // _BR


def _scales_body(histw_ref, hist1_ref, x_ref, xs_ref, aux_ref):
    hw = histw_ref[...]
    h1 = hist1_ref[...]
    degw = hw[0, :, 0] + hw[1, :, 0] + 1.0
    deg1 = h1[0, :, 0] + h1[1, :, 0] + 1.0
    disw = jnp.where(degw > 0, lax.rsqrt(degw), 0.0)
    dis1 = jnp.where(deg1 > 0, lax.rsqrt(deg1), 0.0)
    aux_ref[...] = jnp.stack([disw, disw * disw, dis1, dis1 * dis1], axis=1)
    xs_ref[...] = x_ref[...] * disw[:, None]


def _scales_call(histw, hist1, x):
    return pl.pallas_call(
        _scales_body,
        grid=(_NB,),
        in_specs=[
            pl.BlockSpec((NC, _BR, 16), lambda i: (0, i, 0)),
            pl.BlockSpec((NC, _BR, 16), lambda i: (0, i, 0)),
            pl.BlockSpec((_BR, 128), lambda i: (i, 0)),
        ],
        out_specs=[
            pl.BlockSpec((_BR, 128), lambda i: (i, 0)),
            pl.BlockSpec((_BR, 4), lambda i: (i, 0)),
        ],
        out_shape=[
            jax.ShapeDtypeStruct((N, 128), jnp.float32),
            jax.ShapeDtypeStruct((N, 4), jnp.float32),
        ],
    )(histw, hist1, x)


def _make_conv_body(col, relu, scale_col):
    def body(p_ref, xin_ref, aux_ref, w_ref, b_ref, out_ref, *scaled_ref):
        aux = aux_ref[...]
        t = ((p_ref[0] + p_ref[1]) * aux[:, col][:, None]
             + xin_ref[...] * aux[:, col + 1][:, None])
        o = jnp.dot(t, w_ref[...], preferred_element_type=jnp.float32) + b_ref[0, :]
        if relu:
            o = jnp.maximum(o, 0.0)
        out_ref[...] = o
        if scale_col is not None:
            scaled_ref[0][...] = o * aux[:, scale_col][:, None]

    return body


def _conv_call(p, xin, aux, w, b, col, relu, scale_col):
    kout = w.shape[1]
    out_shape = [jax.ShapeDtypeStruct((N, kout), jnp.float32)]
    out_specs = [pl.BlockSpec((_BR, kout), lambda i: (i, 0))]
    if scale_col is not None:
        out_shape.append(jax.ShapeDtypeStruct((N, kout), jnp.float32))
        out_specs.append(pl.BlockSpec((_BR, kout), lambda i: (i, 0)))
    res = pl.pallas_call(
        _make_conv_body(col, relu, scale_col),
        grid=(_NB,),
        in_specs=[
            pl.BlockSpec((NC, _BR, 128), lambda i: (0, i, 0)),
            pl.BlockSpec((_BR, 128), lambda i: (i, 0)),
            pl.BlockSpec((_BR, 4), lambda i: (i, 0)),
            pl.BlockSpec((128, kout), lambda i: (0, 0)),
            pl.BlockSpec((1, kout), lambda i: (0, 0)),
        ],
        out_specs=out_specs,
        out_shape=out_shape,
    )(p, xin, aux, w, b.reshape(1, kout))
    return res if scale_col is not None else (res[0], None)


def _heads_body(p_ref, xin_ref, aux_ref, wmu_ref, bmu_ref, wls_ref, bls_ref,
                mu_ref, ls_ref):
    aux = aux_ref[...]
    agg = ((p_ref[0] + p_ref[1]) * aux[:, 2][:, None]
           + xin_ref[...] * aux[:, 3][:, None])
    mu_ref[...] = jnp.dot(agg, wmu_ref[...],
                          preferred_element_type=jnp.float32) + bmu_ref[0, :]
    ls_ref[...] = jnp.dot(agg, wls_ref[...],
                          preferred_element_type=jnp.float32) + bls_ref[0, :]


def _heads_call(p, xin, aux, wmu, bmu, wls, bls):
    kout = wmu.shape[1]
    return pl.pallas_call(
        _heads_body,
        grid=(_NB,),
        in_specs=[
            pl.BlockSpec((NC, _BR, 128), lambda i: (0, i, 0)),
            pl.BlockSpec((_BR, 128), lambda i: (i, 0)),
            pl.BlockSpec((_BR, 4), lambda i: (i, 0)),
            pl.BlockSpec((128, kout), lambda i: (0, 0)),
            pl.BlockSpec((1, kout), lambda i: (0, 0)),
            pl.BlockSpec((128, kout), lambda i: (0, 0)),
            pl.BlockSpec((1, kout), lambda i: (0, 0)),
        ],
        out_specs=[
            pl.BlockSpec((_BR, kout), lambda i: (i, 0)),
            pl.BlockSpec((_BR, kout), lambda i: (i, 0)),
        ],
        out_shape=[
            jax.ShapeDtypeStruct((N, kout), jnp.float32),
            jax.ShapeDtypeStruct((N, kout), jnp.float32),
        ],
    )(p, xin, aux, wmu, bmu.reshape(1, kout), wls, bls.reshape(1, kout))


def kernel(x, edge_index, edge_weight, W1, b1, W2, b2, W_mu, b_mu, W_ls, b_ls):
    ei = edge_index.astype(jnp.int32)
    src3 = ei[0].reshape(NW, NCHUNK, K)
    dst3 = ei[1].reshape(NW, NCHUNK, K)
    ew3 = edge_weight.reshape(NW, NCHUNK, K)

    histw, hist1 = _deg_kernel(dst3, ew3)
    xs0, aux = _scales_call(histw, hist1, x)

    p1 = _agg_w(xs0, src3, dst3, ew3)
    h1, h1s = _conv_call(p1, x, aux, W1, b1, col=0, relu=True, scale_col=0)

    p2 = _agg_w(h1s, src3, dst3, ew3)
    h2, h2s = _conv_call(p2, h1, aux, W2, b2, col=0, relu=False, scale_col=2)

    p3 = _agg_1(h2s, src3, dst3, ew3)
    mu, ls = _heads_call(p3, h2, aux, W_mu, b_mu, W_ls, b_ls)
    return (mu, ls)


# speedup vs baseline: 32.9503x; 1.0460x over previous
import functools

import jax
import jax.numpy as jnp
from jax import lax
from jax.experimental import pallas as pl
from jax.experimental.pallas import tpu as pltpu, tpu_sc as plsc

N = 10000
NPAD = 10240
E = 320000
NC = 2
NS = 16
NW = NC * NS
EPW = E // NW
K = 80
NCHUNK = EPW // K
RPW = NPAD // NS
RBLK = K
NRB = RPW // RBLK

_sc_params = pltpu.CompilerParams(use_tc_tiling_on_sc=False)

_mesh = plsc.VectorSubcoreMesh(core_axis_name="c", subcore_axis_name="s",
                               num_cores=NC, num_subcores=NS)


def _zero_block(ref, nrows, ncol16):
    zero16 = jnp.zeros((16,), jnp.float32)

    def body(i, _):
        for q in range(ncol16):
            ref[i, pl.ds(q * 16, 16)] = zero16
        return 0

    lax.fori_loop(0, nrows, body, 0)


@functools.partial(
    pl.kernel,
    out_type=jax.ShapeDtypeStruct((NC, NPAD, 16), jnp.float32),
    mesh=_mesh,
    compiler_params=_sc_params,
    scratch_types=[
        pltpu.VMEM((NCHUNK, K), jnp.int32),
        pltpu.VMEM((NCHUNK, K), jnp.float32),
        pltpu.VMEM((K, 16), jnp.float32),
        pltpu.VMEM((RPW, 16), jnp.float32),
        pltpu.VMEM_SHARED((NPAD, 16), jnp.float32),
    ],
)
def _deg_kernel(dst_hbm, ew_hbm, out_hbm, dst_v, ew_v, buf, zblk, hist):
    c = lax.axis_index("c")
    s = lax.axis_index("s")
    chunk = c * NS + s
    pltpu.sync_copy(dst_hbm.at[chunk], dst_v)
    pltpu.sync_copy(ew_hbm.at[chunk], ew_v)

    _zero_block(zblk, RPW, 1)
    pltpu.sync_copy(zblk, hist.at[pl.ds(s * RPW, RPW)])

    ones_f = jnp.ones((16,), jnp.float32)
    lane_lo = lax.broadcasted_iota(jnp.int32, (16,), 0) < 8
    plsc.subcore_barrier()

    def chunk_body(j, _):
        def grp_body(g, _):
            wv = ew_v[j, pl.ds(g * 16, 16)]
            for l in range(16):
                buf[g * 16 + l, :] = jnp.where(
                    lane_lo, jnp.broadcast_to(wv[l], (16,)), ones_f)
            return 0
        lax.fori_loop(0, K // 16, grp_body, 0)
        pltpu.sync_copy(buf, hist.at[dst_v.at[j]], add=True)
        return 0

    lax.fori_loop(0, NCHUNK, chunk_body, 0)
    plsc.subcore_barrier()

    pltpu.sync_copy(hist.at[pl.ds(s * RPW, RPW)],
                    out_hbm.at[c, pl.ds(s * RPW, RPW)])


def _make_agg_kernel(weighted):
    scratch = [
        pltpu.VMEM((NCHUNK, K), jnp.int32),
        pltpu.VMEM((NCHUNK, K), jnp.int32),
        pltpu.VMEM((K, 128), jnp.float32),
        pltpu.VMEM((K, 128), jnp.float32),
        pltpu.VMEM((1, K), jnp.float32),
        pltpu.VMEM((1, K), jnp.float32),
        pltpu.VMEM_SHARED((NPAD, 128), jnp.float32),
        pltpu.SemaphoreType.DMA,
        pltpu.SemaphoreType.DMA,
        pltpu.SemaphoreType.DMA,
        pltpu.SemaphoreType.DMA,
    ]

    def body(table_hbm, src_hbm, dst_hbm, ew_hbm, out_hbm,
             src_v, dst_v, rows_a, rows_b, ew_a, ew_b, acc,
             sem_ra, sem_rb, sem_wa, sem_wb):
        c = lax.axis_index("c")
        s = lax.axis_index("s")
        chunk = c * NS + s
        pltpu.sync_copy(src_hbm.at[chunk], src_v)
        pltpu.sync_copy(dst_hbm.at[chunk], dst_v)

        _zero_block(rows_a, RBLK, 8)
        for t in range(NRB):
            pltpu.sync_copy(rows_a, acc.at[pl.ds(s * RPW + t * RBLK, RBLK)])
        plsc.subcore_barrier()

        def fetch(j, rows, ew, sem_r, sem_w):
            pltpu.async_copy(table_hbm.at[src_v.at[j]], rows, sem_r)
            if weighted:
                pltpu.async_copy(ew_hbm.at[chunk].at[pl.ds(j, 1)], ew, sem_w)

        def wait(rows, ew, sem_r, sem_w):
            pltpu.make_async_copy(table_hbm.at[src_v.at[0]], rows, sem_r).wait()
            if weighted:
                pltpu.make_async_copy(ew_hbm.at[0].at[pl.ds(0, 1)], ew,
                                      sem_w).wait()

        def process(j, rows, ew):
            if weighted:
                def grp_body(g, _):
                    wv = ew[0, pl.ds(g * 16, 16)]
                    for l in range(16):
                        w = wv[l]
                        r = g * 16 + l
                        for q in range(8):
                            sl = pl.ds(q * 16, 16)
                            rows[r, sl] = rows[r, sl] * w
                    return 0
                lax.fori_loop(0, K // 16, grp_body, 0)
            pltpu.sync_copy(rows, acc.at[dst_v.at[j]], add=True)

        fetch(0, rows_a, ew_a, sem_ra, sem_wa)

        def pair_body(i, _):
            c0 = 2 * i
            fetch(c0 + 1, rows_b, ew_b, sem_rb, sem_wb)
            wait(rows_a, ew_a, sem_ra, sem_wa)
            process(c0, rows_a, ew_a)
            fetch(c0 + 2, rows_a, ew_a, sem_ra, sem_wa)
            wait(rows_b, ew_b, sem_rb, sem_wb)
            process(c0 + 1, rows_b, ew_b)
            return 0

        lax.fori_loop(0, (NCHUNK - 1) // 2, pair_body, 0)
        wait(rows_a, ew_a, sem_ra, sem_wa)
        process(NCHUNK - 1, rows_a, ew_a)
        plsc.subcore_barrier()

        pltpu.sync_copy(acc.at[pl.ds(s * RPW, RPW)],
                        out_hbm.at[c, pl.ds(s * RPW, RPW)])

    return pl.kernel(
        body,
        out_type=jax.ShapeDtypeStruct((NC, NPAD, 128), jnp.float32),
        mesh=_mesh,
        compiler_params=_sc_params,
        scratch_types=scratch,
    )


_agg_w = _make_agg_kernel(True)
_agg_1 = _make_agg_kernel(False)


_BR = 1000
_NB = N // _BR


def _scales_body(hist_ref, x_ref, xs_ref, aux_ref):
    h = hist_ref[...]
    degw = h[0, :, 0] + h[1, :, 0] + 1.0
    deg1 = h[0, :, 8] + h[1, :, 8] + 1.0
    disw = jnp.where(degw > 0, lax.rsqrt(degw), 0.0)
    dis1 = jnp.where(deg1 > 0, lax.rsqrt(deg1), 0.0)
    aux_ref[...] = jnp.stack([disw, disw * disw, dis1, dis1 * dis1], axis=1)
    xs_ref[...] = x_ref[...] * disw[:, None]


def _scales_call(hist, x):
    return pl.pallas_call(
        _scales_body,
        grid=(_NB,),
        in_specs=[
            pl.BlockSpec((NC, _BR, 16), lambda i: (0, i, 0)),
            pl.BlockSpec((_BR, 128), lambda i: (i, 0)),
        ],
        out_specs=[
            pl.BlockSpec((_BR, 128), lambda i: (i, 0)),
            pl.BlockSpec((_BR, 4), lambda i: (i, 0)),
        ],
        out_shape=[
            jax.ShapeDtypeStruct((N, 128), jnp.float32),
            jax.ShapeDtypeStruct((N, 4), jnp.float32),
        ],
    )(hist, x)


def _make_conv_body(col, relu, scale_col):
    def body(p_ref, xin_ref, aux_ref, w_ref, b_ref, out_ref, *scaled_ref):
        aux = aux_ref[...]
        t = ((p_ref[0] + p_ref[1]) * aux[:, col][:, None]
             + xin_ref[...] * aux[:, col + 1][:, None])
        o = jnp.dot(t, w_ref[...], preferred_element_type=jnp.float32) + b_ref[0, :]
        if relu:
            o = jnp.maximum(o, 0.0)
        out_ref[...] = o
        if scale_col is not None:
            scaled_ref[0][...] = o * aux[:, scale_col][:, None]

    return body


def _conv_call(p, xin, aux, w, b, col, relu, scale_col):
    kout = w.shape[1]
    out_shape = [jax.ShapeDtypeStruct((N, kout), jnp.float32)]
    out_specs = [pl.BlockSpec((_BR, kout), lambda i: (i, 0))]
    if scale_col is not None:
        out_shape.append(jax.ShapeDtypeStruct((N, kout), jnp.float32))
        out_specs.append(pl.BlockSpec((_BR, kout), lambda i: (i, 0)))
    res = pl.pallas_call(
        _make_conv_body(col, relu, scale_col),
        grid=(_NB,),
        in_specs=[
            pl.BlockSpec((NC, _BR, 128), lambda i: (0, i, 0)),
            pl.BlockSpec((_BR, 128), lambda i: (i, 0)),
            pl.BlockSpec((_BR, 4), lambda i: (i, 0)),
            pl.BlockSpec((128, kout), lambda i: (0, 0)),
            pl.BlockSpec((1, kout), lambda i: (0, 0)),
        ],
        out_specs=out_specs,
        out_shape=out_shape,
    )(p, xin, aux, w, b.reshape(1, kout))
    return res if scale_col is not None else (res[0], None)


def _heads_body(p_ref, xin_ref, aux_ref, wmu_ref, bmu_ref, wls_ref, bls_ref,
                mu_ref, ls_ref):
    aux = aux_ref[...]
    agg = ((p_ref[0] + p_ref[1]) * aux[:, 2][:, None]
           + xin_ref[...] * aux[:, 3][:, None])
    mu_ref[...] = jnp.dot(agg, wmu_ref[...],
                          preferred_element_type=jnp.float32) + bmu_ref[0, :]
    ls_ref[...] = jnp.dot(agg, wls_ref[...],
                          preferred_element_type=jnp.float32) + bls_ref[0, :]


def _heads_call(p, xin, aux, wmu, bmu, wls, bls):
    kout = wmu.shape[1]
    return pl.pallas_call(
        _heads_body,
        grid=(_NB,),
        in_specs=[
            pl.BlockSpec((NC, _BR, 128), lambda i: (0, i, 0)),
            pl.BlockSpec((_BR, 128), lambda i: (i, 0)),
            pl.BlockSpec((_BR, 4), lambda i: (i, 0)),
            pl.BlockSpec((128, kout), lambda i: (0, 0)),
            pl.BlockSpec((1, kout), lambda i: (0, 0)),
            pl.BlockSpec((128, kout), lambda i: (0, 0)),
            pl.BlockSpec((1, kout), lambda i: (0, 0)),
        ],
        out_specs=[
            pl.BlockSpec((_BR, kout), lambda i: (i, 0)),
            pl.BlockSpec((_BR, kout), lambda i: (i, 0)),
        ],
        out_shape=[
            jax.ShapeDtypeStruct((N, kout), jnp.float32),
            jax.ShapeDtypeStruct((N, kout), jnp.float32),
        ],
    )(p, xin, aux, wmu, bmu.reshape(1, kout), wls, bls.reshape(1, kout))


def kernel(x, edge_index, edge_weight, W1, b1, W2, b2, W_mu, b_mu, W_ls, b_ls):
    ei = edge_index.astype(jnp.int32)
    src3 = ei[0].reshape(NW, NCHUNK, K)
    dst3 = ei[1].reshape(NW, NCHUNK, K)
    ew3 = edge_weight.reshape(NW, NCHUNK, K)

    hist = _deg_kernel(dst3, ew3)
    xs0, aux = _scales_call(hist, x)

    p1 = _agg_w(xs0, src3, dst3, ew3)
    h1, h1s = _conv_call(p1, x, aux, W1, b1, col=0, relu=True, scale_col=0)

    p2 = _agg_w(h1s, src3, dst3, ew3)
    h2, h2s = _conv_call(p2, h1, aux, W2, b2, col=0, relu=False, scale_col=2)

    p3 = _agg_1(h2s, src3, dst3, ew3)
    mu, ls = _heads_call(p3, h2, aux, W_mu, b_mu, W_ls, b_ls)
    return (mu, ls)
